# Initial kernel scaffold; baseline (speedup 1.0000x reference)
import jax
import jax.numpy as jnp
from jax.experimental import pallas as pl


def kernel(x, emb_weight):
    raise NotImplementedError("write your pallas kernel here")


# speedup vs baseline: 7.7502x; 7.7502x over previous
import functools

import jax
import jax.numpy as jnp
from jax import lax
from jax.experimental import pallas as pl
from jax.experimental.pallas import tpu as pltpu
from jax.experimental.pallas import tpu_sc as plsc

_NC = 2
_NS = 16
_NW = _NC * _NS

_CHUNK = 2000


@functools.partial(jax.jit, static_argnames=())
def kernel(x, emb_weight):
    n = x.shape[0]
    hidden = emb_weight.shape[1]
    assert n % _NW == 0
    per_w = n // _NW
    assert per_w % _CHUNK == 0
    n_chunks = per_w // _CHUNK

    mesh = plsc.VectorSubcoreMesh(core_axis_name="c", subcore_axis_name="s")

    @functools.partial(
        pl.kernel,
        out_type=jax.ShapeDtypeStruct((n, hidden), jnp.float32),
        mesh=mesh,
        scratch_types=[
            pltpu.VMEM((_CHUNK,), jnp.int32),
            pltpu.VMEM((_CHUNK, hidden), jnp.float32),
            pltpu.SemaphoreType.DMA,
        ],
        compiler_params=pltpu.CompilerParams(use_tc_tiling_on_sc=False),
    )
    def _run(x_hbm, tab_hbm, out_hbm, idx_v, rows_v, sem):
        wid = lax.axis_index("s") * _NC + lax.axis_index("c")
        base = wid * per_w

        def step(k, carry):
            off = pl.multiple_of(base + k * _CHUNK, 8)
            pltpu.sync_copy(x_hbm.at[pl.ds(off, _CHUNK)], idx_v)
            pltpu.async_copy(tab_hbm.at[idx_v], rows_v, sem).wait()
            pltpu.sync_copy(rows_v, out_hbm.at[pl.ds(off, _CHUNK)])
            return carry

        lax.fori_loop(0, n_chunks, step, 0, unroll=False)

    return _run(x.astype(jnp.int32), emb_weight)


# speedup vs baseline: 13.7430x; 1.7732x over previous
import functools

import jax
import jax.numpy as jnp
from jax import lax
from jax.experimental import pallas as pl
from jax.experimental.pallas import tpu as pltpu
from jax.experimental.pallas import tpu_sc as plsc

_NC = 2
_NS = 16
_NW = _NC * _NS

_TCOLS = 10
_CIDX = _TCOLS * 128


def kernel(x, emb_weight):
    n = x.shape[0]
    v, hidden = emb_weight.shape
    assert hidden == 16 and n % (_CIDX) == 0
    n_tc = n // 128
    n_chunks = n_tc // _TCOLS

    mesh = plsc.VectorSubcoreMesh(core_axis_name="c", subcore_axis_name="s")

    @functools.partial(
        pl.kernel,
        out_type=jax.ShapeDtypeStruct((2, n_tc, 8, 128), jnp.float32),
        mesh=mesh,
        scratch_types=[
            pltpu.VMEM((v, hidden), jnp.float32),
            pltpu.VMEM((_CIDX,), jnp.int32),
            pltpu.VMEM((2, _TCOLS, 8, 128), jnp.float32),
            pltpu.SemaphoreType.DMA,
        ],
        compiler_params=pltpu.CompilerParams(
            use_tc_tiling_on_sc=False, needs_layout_passes=False),
    )
    def _run(x_hbm, tab_hbm, out_hbm, tab_v, idx_v, buf_v, sem):
        wid = lax.axis_index("s") * _NC + lax.axis_index("c")
        pltpu.sync_copy(tab_hbm, tab_v)
        n_mine = (n_chunks - wid + _NW - 1) // _NW

        def step(k, carry):
            cid = wid + k * _NW
            ioff = pl.multiple_of(cid * _CIDX, 8)
            pltpu.sync_copy(x_hbm.at[pl.ds(ioff, _CIDX)], idx_v)
            for j in range(_CIDX // 16):
                xv = idx_v[pl.ds(16 * j, 16)]
                tcb, lane0 = j // 8, (j % 8) * 16
                for h in range(16):
                    vals = plsc.load_gather(
                        tab_v, [xv, jnp.full((16,), h, jnp.int32)])
                    buf_v[h // 8, tcb, h % 8, pl.ds(lane0, 16)] = vals
            pltpu.sync_copy(buf_v, out_hbm.at[:, pl.ds(cid * _TCOLS, _TCOLS)])
            return carry

        lax.fori_loop(0, n_mine, step, 0, unroll=False)

    out4 = _run(x.astype(jnp.int32), emb_weight)
    return out4.transpose(1, 3, 0, 2).reshape(n, hidden)


# speedup vs baseline: 93.1106x; 6.7751x over previous
import functools

import jax
import jax.numpy as jnp
from jax import lax
from jax.experimental import pallas as pl
from jax.experimental.pallas import tpu as pltpu
from jax.experimental.pallas import tpu_sc as plsc

_NC = 2
_NS = 16
_NW = _NC * _NS

_TCOLS = 10
_CIDX = _TCOLS * 128


def kernel(x, emb_weight):
    n = x.shape[0]
    v, hidden = emb_weight.shape
    assert hidden == 16 and n % (_CIDX) == 0
    n_tc = n // 128
    n_chunks = n_tc // _TCOLS

    mesh = plsc.VectorSubcoreMesh(core_axis_name="c", subcore_axis_name="s")

    @functools.partial(
        pl.kernel,
        out_type=jax.ShapeDtypeStruct((2, n_tc, 8, 128), jnp.float32),
        mesh=mesh,
        scratch_types=[
            pltpu.VMEM((v, hidden), jnp.float32),
            pltpu.VMEM((_CIDX,), jnp.int32),
            pltpu.VMEM((2, _TCOLS, 8, 128), jnp.float32),
            pltpu.SemaphoreType.DMA,
        ],
        compiler_params=pltpu.CompilerParams(
            use_tc_tiling_on_sc=False, needs_layout_passes=False),
    )
    def _run(x_hbm, tab_hbm, out_hbm, tab_v, idx_v, buf_v, sem):
        wid = lax.axis_index("s") * _NC + lax.axis_index("c")
        pltpu.sync_copy(tab_hbm, tab_v)
        n_mine = (n_chunks - wid + _NW - 1) // _NW

        def step(k, carry):
            cid = wid + k * _NW
            ioff = pl.multiple_of(cid * _CIDX, 8)
            pltpu.sync_copy(x_hbm.at[pl.ds(ioff, _CIDX)], idx_v)
            for tcb in range(_TCOLS):
                @functools.partial(plsc.parallel_loop, 0, 8, unroll=2)
                def _gather(jj, _tcb=tcb):
                    xv = idx_v[pl.ds(_tcb * 128 + jj * 16, 16)]
                    for h in range(16):
                        vals = plsc.load_gather(
                            tab_v, [xv, jnp.full((16,), h, jnp.int32)])
                        buf_v[h // 8, _tcb, h % 8, pl.ds(jj * 16, 16)] = vals
            pltpu.sync_copy(buf_v, out_hbm.at[:, pl.ds(cid * _TCOLS, _TCOLS)])
            return carry

        lax.fori_loop(0, n_mine, step, 0, unroll=False)

    out4 = _run(x.astype(jnp.int32), emb_weight)
    return out4.transpose(1, 3, 0, 2).reshape(n, hidden)
